# baseline (speedup 1.0000x reference)
import functools

import jax
import jax.numpy as jnp
from jax.experimental import pallas as pl

N, D, DE, DT = 10000, 128, 32, 10
BLOCK = 10000


def _body(x_ref, wz_ref, bz_ref, wh_ref, bh_ref, blin_ref, wp_ref, bp_ref,
          out_ref, h_ref):
    x = x_ref[...]
    az = wz_ref[0] + wz_ref[1]
    ah = wh_ref[0] + wh_ref[1]
    z = jax.nn.sigmoid(
        jnp.dot(x, az, preferred_element_type=jnp.float32) + bz_ref[...])
    ht = jnp.tanh(
        jnp.dot(x, ah, preferred_element_type=jnp.float32) + bh_ref[...])
    h = jnp.maximum((1.0 - z) * ht + blin_ref[...], 0.0)
    h_ref[...] = h
    out_ref[...] = (
        jnp.dot(h, wp_ref[...], preferred_element_type=jnp.float32)
        + bp_ref[...])


@functools.partial(jax.jit, static_argnames=())
def kernel(x, edge_index, mask, W_z, b_z, W_r, b_r, W_h, b_h,
           W_lin, b_lin, W_pred, b_pred):
    del edge_index, mask, W_r, b_r, W_lin
    wz = W_z[:, 0, :D, :]
    wh = W_h[:, 0, :D, :]
    bz = b_z.reshape(1, DE)
    bh = b_h.reshape(1, DE)
    blin = b_lin.reshape(1, DE)
    bp = b_pred.reshape(1, DT)

    grid = (N // BLOCK,)
    out, h = pl.pallas_call(
        _body,
        grid=grid,
        in_specs=[
            pl.BlockSpec((BLOCK, D), lambda i: (i, 0)),
            pl.BlockSpec((2, D, DE), lambda i: (0, 0, 0)),
            pl.BlockSpec((1, DE), lambda i: (0, 0)),
            pl.BlockSpec((2, D, DE), lambda i: (0, 0, 0)),
            pl.BlockSpec((1, DE), lambda i: (0, 0)),
            pl.BlockSpec((1, DE), lambda i: (0, 0)),
            pl.BlockSpec((DE, DT), lambda i: (0, 0)),
            pl.BlockSpec((1, DT), lambda i: (0, 0)),
        ],
        out_specs=[
            pl.BlockSpec((BLOCK, DT), lambda i: (i, 0)),
            pl.BlockSpec((BLOCK, DE), lambda i: (i, 0)),
        ],
        out_shape=[
            jax.ShapeDtypeStruct((N, DT), jnp.float32),
            jax.ShapeDtypeStruct((N, DE), jnp.float32),
        ],
    )(x, wz, bz, wh, bh, blin, W_pred, bp)
    return (out, h)


# speedup vs baseline: 1.0803x; 1.0803x over previous
import functools

import jax
import jax.numpy as jnp
from jax.experimental import pallas as pl

N, D, DE, DT = 10000, 128, 32, 10
BLOCK = 5000


def _body(x_ref, wz_ref, bz_ref, wh_ref, bh_ref, blin_ref, wp_ref, bp_ref,
          out_ref, h_ref):
    x = x_ref[...]
    w = jnp.concatenate(
        [wz_ref[0] + wz_ref[1], wh_ref[0] + wh_ref[1]], axis=1)
    g = jnp.dot(x, w, preferred_element_type=jnp.float32)
    z = jax.nn.sigmoid(g[:, :DE] + bz_ref[...])
    ht = jnp.tanh(g[:, DE:] + bh_ref[...])
    h = jnp.maximum((1.0 - z) * ht + blin_ref[...], 0.0)
    h_ref[...] = h
    out_ref[...] = (
        jnp.dot(h, wp_ref[...], preferred_element_type=jnp.float32)
        + bp_ref[...])


@functools.partial(jax.jit, static_argnames=())
def kernel(x, edge_index, mask, W_z, b_z, W_r, b_r, W_h, b_h,
           W_lin, b_lin, W_pred, b_pred):
    del edge_index, mask, W_r, b_r, W_lin
    wz = W_z[:, 0, :D, :]
    wh = W_h[:, 0, :D, :]
    bz = b_z.reshape(1, DE)
    bh = b_h.reshape(1, DE)
    blin = b_lin.reshape(1, DE)
    bp = b_pred.reshape(1, DT)

    grid = (N // BLOCK,)
    out, h = pl.pallas_call(
        _body,
        grid=grid,
        in_specs=[
            pl.BlockSpec((BLOCK, D), lambda i: (i, 0)),
            pl.BlockSpec((2, D, DE), lambda i: (0, 0, 0)),
            pl.BlockSpec((1, DE), lambda i: (0, 0)),
            pl.BlockSpec((2, D, DE), lambda i: (0, 0, 0)),
            pl.BlockSpec((1, DE), lambda i: (0, 0)),
            pl.BlockSpec((1, DE), lambda i: (0, 0)),
            pl.BlockSpec((DE, DT), lambda i: (0, 0)),
            pl.BlockSpec((1, DT), lambda i: (0, 0)),
        ],
        out_specs=[
            pl.BlockSpec((BLOCK, DT), lambda i: (i, 0)),
            pl.BlockSpec((BLOCK, DE), lambda i: (i, 0)),
        ],
        out_shape=[
            jax.ShapeDtypeStruct((N, DT), jnp.float32),
            jax.ShapeDtypeStruct((N, DE), jnp.float32),
        ],
    )(x, wz, bz, wh, bh, blin, W_pred, bp)
    return (out, h)
